# Initial kernel scaffold; baseline (speedup 1.0000x reference)
import jax
import jax.numpy as jnp
from jax.experimental import pallas as pl


def kernel(X, tables):
    raise NotImplementedError("write your pallas kernel here")


# speedup vs baseline: 1.1532x; 1.1532x over previous
import functools

import jax
import jax.numpy as jnp
from jax import lax
from jax.experimental import pallas as pl
from jax.experimental.pallas import tpu as pltpu
from jax.experimental.pallas import tpu_sc as plsc

_F = 26
_V = 100000
_D = 32
_B = 16384

_NW = 32
_ROWS = _B * _F
_RPW = _ROWS // _NW
_IDXW = 128
_CHUNK = 1024
_NCHUNK = _RPW // _CHUNK
_SPC = _CHUNK // _IDXW

_mesh = plsc.VectorSubcoreMesh(core_axis_name="c", subcore_axis_name="s")


@functools.partial(
    pl.kernel,
    mesh=_mesh,
    compiler_params=pltpu.CompilerParams(use_tc_tiling_on_sc=False),
    out_type=jax.ShapeDtypeStruct((_ROWS, _D), jnp.float32),
    scratch_types=[
        pltpu.VMEM((_RPW // _IDXW, _IDXW), jnp.int32),
        pltpu.VMEM((_CHUNK, _D), jnp.float32),
        pltpu.VMEM((_CHUNK, _D), jnp.float32),
        pltpu.SemaphoreType.DMA,
        pltpu.SemaphoreType.DMA,
        pltpu.SemaphoreType.DMA,
        pltpu.SemaphoreType.DMA,
    ],
)
def _gather_kernel(table_hbm, idx_hbm, out_hbm,
                   idx_v, rows0, rows1, gsem0, gsem1, osem0, osem1):
    wid = lax.axis_index("s") * 2 + lax.axis_index("c")
    base = wid * _RPW

    rows = (rows0, rows1)
    gsem = (gsem0, gsem1)
    osem = (osem0, osem1)

    pltpu.sync_copy(idx_hbm.at[pl.ds(wid * (_RPW // _IDXW), _RPW // _IDXW)],
                    idx_v)

    def start_gather(c):
        b = c & 1
        descs = []
        for r in range(_SPC):
            row = c * _SPC + r
            descs.append(pltpu.async_copy(
                table_hbm.at[idx_v.at[row]],
                rows[b].at[pl.ds(r * _IDXW, _IDXW)],
                gsem[b],
            ))
        return descs

    gwait = [None] * _NCHUNK
    owait = [None] * _NCHUNK

    gwait[0] = start_gather(0)
    for c in range(_NCHUNK):
        b = c & 1
        if c + 1 < _NCHUNK:
            if c >= 1:
                owait[c - 1].wait()
            gwait[c + 1] = start_gather(c + 1)
        for d in gwait[c]:
            d.wait()
        owait[c] = pltpu.async_copy(
            rows[b],
            out_hbm.at[pl.ds(base + c * _CHUNK, _CHUNK)],
            osem[b],
        )
    owait[_NCHUNK - 2].wait()
    owait[_NCHUNK - 1].wait()


def kernel(X, tables):
    offs = (jnp.arange(_F, dtype=jnp.int32) * _V)[None, :]
    flat_idx = (X.astype(jnp.int32) + offs).reshape(_ROWS // _IDXW, _IDXW)
    table2d = tables.reshape(_F * _V, _D)
    out = _gather_kernel(table2d, flat_idx)
    return out.reshape(_B, _F, _D)
